# baseline (speedup 1.0000x reference)
import functools

import jax
import jax.numpy as jnp
from jax import lax
from jax.experimental import pallas as pl
from jax.experimental.pallas import tpu as pltpu
from jax.experimental.pallas import tpu_sc as plsc

CHUNK = 128
NC = 2
NS = 16
BM = 256


def _sc_mesh():
    return plsc.VectorSubcoreMesh(core_axis_name="c", subcore_axis_name="s")


@functools.lru_cache(maxsize=None)
def _make_sc_deg(nchunks, n_pad, h):
    rows = n_pad // NS

    @functools.partial(
        pl.kernel,
        mesh=_sc_mesh(),
        out_type=jax.ShapeDtypeStruct((NC * n_pad, h), jnp.float32),
        scratch_types=[
            pltpu.VMEM((nchunks, CHUNK), jnp.int32),
            pltpu.VMEM((CHUNK, h), jnp.float32),
            pltpu.VMEM_SHARED((n_pad, h), jnp.float32),
            pltpu.SemaphoreType.DMA,
            pltpu.SemaphoreType.DMA,
        ],
    )
    def deg_kernel(dst_hbm, ones_hbm, zero_hbm, out_hbm,
                   didx, ones_v, acc, sem0, sem1):
        c = lax.axis_index("c")
        s = lax.axis_index("s")
        wid = c * NS + s
        r0 = s * rows
        pltpu.sync_copy(ones_hbm, ones_v)
        pltpu.sync_copy(dst_hbm.at[wid], didx)
        pltpu.sync_copy(zero_hbm.at[pl.ds(r0, rows), :], acc.at[pl.ds(r0, rows), :])
        plsc.subcore_barrier()

        def body(i, carry):
            g0 = 2 * i
            pltpu.async_copy(ones_v, acc.at[didx.at[g0]], sem0, add=True)
            pltpu.async_copy(ones_v, acc.at[didx.at[g0 + 1]], sem1, add=True)
            pltpu.make_async_copy(ones_v, acc.at[didx.at[g0]], sem0).wait()
            pltpu.make_async_copy(ones_v, acc.at[didx.at[g0 + 1]], sem1).wait()
            return carry

        lax.fori_loop(0, nchunks // 2, body, 0)
        plsc.subcore_barrier()
        pltpu.sync_copy(acc.at[pl.ds(r0, rows), :],
                        out_hbm.at[pl.ds(c * n_pad + r0, rows), :])

    return deg_kernel


@functools.lru_cache(maxsize=None)
def _make_sc_scatter(nchunks, n_pad, h):
    rows = n_pad // NS

    @functools.partial(
        pl.kernel,
        mesh=_sc_mesh(),
        out_type=jax.ShapeDtypeStruct((NC * n_pad, h), jnp.float32),
        scratch_types=[
            pltpu.VMEM((CHUNK,), jnp.int32),
            pltpu.VMEM((CHUNK,), jnp.int32),
            pltpu.VMEM((CHUNK, h), jnp.float32),
            pltpu.VMEM_SHARED((n_pad, h), jnp.float32),
            pltpu.SemaphoreType.DMA,
        ],
    )
    def scatter_kernel(y_hbm, src_hbm, dst_hbm, zero_hbm, out_hbm,
                       sidx, didx, gbuf, acc, sem):
        c = lax.axis_index("c")
        s = lax.axis_index("s")
        wid = c * NS + s
        r0 = s * rows
        pltpu.sync_copy(zero_hbm.at[pl.ds(r0, rows), :], acc.at[pl.ds(r0, rows), :])
        plsc.subcore_barrier()

        def body(g, carry):
            pltpu.sync_copy(src_hbm.at[wid, g], sidx)
            pltpu.sync_copy(dst_hbm.at[wid, g], didx)
            pltpu.async_copy(y_hbm.at[sidx], gbuf, sem).wait()
            pltpu.sync_copy(gbuf, acc.at[didx], add=True)
            return carry

        lax.fori_loop(0, nchunks, body, 0)
        plsc.subcore_barrier()
        pltpu.sync_copy(acc.at[pl.ds(r0, rows), :],
                        out_hbm.at[pl.ds(c * n_pad + r0, rows), :])

    return scatter_kernel


def _tc_prep(x_pad, w1, degp):
    n_pad, f = x_pad.shape
    h = w1.shape[1]
    nb = n_pad // BM

    def body(x_ref, w_ref, d0_ref, d1_ref, y_ref, dinv_ref):
        deg = jnp.maximum(d0_ref[...][:, 0:1] + d1_ref[...][:, 0:1], 1.0)
        dinv = lax.rsqrt(deg)
        xw = jnp.dot(x_ref[...], w_ref[...],
                     preferred_element_type=jnp.float32)
        y_ref[...] = xw * dinv
        dinv_ref[...] = dinv

    return pl.pallas_call(
        body,
        grid=(nb,),
        in_specs=[
            pl.BlockSpec((BM, f), lambda i: (i, 0)),
            pl.BlockSpec((f, h), lambda i: (0, 0)),
            pl.BlockSpec((BM, h), lambda i: (i, 0)),
            pl.BlockSpec((BM, h), lambda i: (i + nb, 0)),
        ],
        out_specs=[
            pl.BlockSpec((BM, h), lambda i: (i, 0)),
            pl.BlockSpec((BM, 1), lambda i: (i, 0)),
        ],
        out_shape=[
            jax.ShapeDtypeStruct((n_pad, h), jnp.float32),
            jax.ShapeDtypeStruct((n_pad, 1), jnp.float32),
        ],
    )(x_pad, w1, degp, degp)


def _tc_mid(p, dinv, b1, w2):
    n_pad = p.shape[0] // NC
    h = p.shape[1]
    nb = n_pad // BM

    def body(p0_ref, p1_ref, dinv_ref, b_ref, w_ref, y2_ref):
        t = dinv_ref[...] * (p0_ref[...] + p1_ref[...]) + b_ref[...]
        hh = jnp.maximum(t, 0.0)
        y2_ref[...] = jnp.dot(hh, w_ref[...],
                              preferred_element_type=jnp.float32) * dinv_ref[...]

    return pl.pallas_call(
        body,
        grid=(nb,),
        in_specs=[
            pl.BlockSpec((BM, h), lambda i: (i, 0)),
            pl.BlockSpec((BM, h), lambda i: (i + nb, 0)),
            pl.BlockSpec((BM, 1), lambda i: (i, 0)),
            pl.BlockSpec((1, h), lambda i: (0, 0)),
            pl.BlockSpec((h, h), lambda i: (0, 0)),
        ],
        out_specs=pl.BlockSpec((BM, h), lambda i: (i, 0)),
        out_shape=jax.ShapeDtypeStruct((n_pad, h), jnp.float32),
    )(p, p, dinv, b1, w2)


def _tc_final(q, dinv, b2, batch_row, wlin, blin, g):
    n_pad = q.shape[0] // NC
    h = q.shape[1]
    nb = n_pad // BM

    def body(q0_ref, q1_ref, dinv_ref, b_ref, batch_ref, wlin_ref, blin_ref,
             out_ref, s_acc, c_acc):
        i = pl.program_id(0)

        @pl.when(i == 0)
        def _():
            s_acc[...] = jnp.zeros_like(s_acc)
            c_acc[...] = jnp.zeros_like(c_acc)

        t = dinv_ref[...] * (q0_ref[...] + q1_ref[...]) + b_ref[...]
        h2 = jnp.maximum(t, 0.0)
        seg = lax.broadcasted_iota(jnp.int32, (g, BM), 0)
        oh = (seg == batch_ref[...]).astype(jnp.float32)
        s_acc[...] += lax.dot_general(
            oh, h2, (((1,), (0,)), ((), ())),
            preferred_element_type=jnp.float32,
            precision=lax.Precision.HIGHEST)
        c_acc[...] += jnp.sum(oh, axis=1)[:, None]

        @pl.when(i == nb - 1)
        def _():
            pooled = s_acc[...] / jnp.maximum(c_acc[...], 1.0)
            out_ref[...] = jnp.dot(pooled, wlin_ref[...],
                                   preferred_element_type=jnp.float32) + blin_ref[...]

    return pl.pallas_call(
        body,
        grid=(nb,),
        in_specs=[
            pl.BlockSpec((BM, h), lambda i: (i, 0)),
            pl.BlockSpec((BM, h), lambda i: (i + nb, 0)),
            pl.BlockSpec((BM, 1), lambda i: (i, 0)),
            pl.BlockSpec((1, h), lambda i: (0, 0)),
            pl.BlockSpec((1, BM), lambda i: (0, i)),
            pl.BlockSpec((h, 1), lambda i: (0, 0)),
            pl.BlockSpec((1, 1), lambda i: (0, 0)),
        ],
        out_specs=pl.BlockSpec((g, 1), lambda i: (0, 0)),
        out_shape=jax.ShapeDtypeStruct((g, 1), jnp.float32),
        scratch_shapes=[
            pltpu.VMEM((g, h), jnp.float32),
            pltpu.VMEM((g, 1), jnp.float32),
        ],
    )(q, q, dinv, b2, batch_row, wlin, blin)


def _round_up(a, b):
    return -(-a // b) * b


def kernel(x, edge_index, batch, W1, b1, W2, b2, Wlin, blin):
    n, f = x.shape
    h = W1.shape[1]
    e = edge_index.shape[1]
    g = 64
    nt = NC * NS

    n_pad = _round_up(n + 1, BM)
    e2 = e + n
    per_tile = _round_up(-(-e2 // nt), 2 * CHUNK)
    ep = per_tile * nt
    nchunks = per_tile // CHUNK

    idt = jnp.int32
    loop = jnp.arange(n, dtype=idt)
    pad = ep - e2
    src_pad = jnp.concatenate(
        [edge_index[0].astype(idt), loop, jnp.zeros((pad,), idt)]
    ).reshape(nt, nchunks, CHUNK)
    dst_fill = n + jnp.arange(pad, dtype=idt) % (n_pad - n)
    dst_pad = jnp.concatenate(
        [edge_index[1].astype(idt), loop, dst_fill]
    ).reshape(nt, nchunks, CHUNK)

    x_pad = jnp.pad(x, ((0, n_pad - n), (0, 0)))
    batch_row = jnp.pad(batch.astype(idt), (0, n_pad - n), constant_values=g)[None, :]
    ones128 = jnp.ones((CHUNK, h), jnp.float32)
    z128 = jnp.zeros((n_pad, h), jnp.float32)

    degp = _make_sc_deg(nchunks, n_pad, h)(dst_pad, ones128, z128)
    y1, dinv = _tc_prep(x_pad, W1, degp)

    scat = _make_sc_scatter(nchunks, n_pad, h)
    p = scat(y1, src_pad, dst_pad, z128)
    y2 = _tc_mid(p, dinv, b1.reshape(1, h), W2)
    q = scat(y2, src_pad, dst_pad, z128)
    return _tc_final(q, dinv, b2.reshape(1, h), batch_row, Wlin, blin.reshape(1, 1), g)


# speedup vs baseline: 1.4691x; 1.4691x over previous
import functools

import jax
import jax.numpy as jnp
from jax import lax
from jax.experimental import pallas as pl
from jax.experimental.pallas import tpu as pltpu
from jax.experimental.pallas import tpu_sc as plsc

CHUNK = 128
NC = 2
NS = 16
BM = 256


def _sc_mesh():
    return plsc.VectorSubcoreMesh(core_axis_name="c", subcore_axis_name="s")


@functools.lru_cache(maxsize=None)
def _make_sc_deg(nchunks, n_pad, h):
    rows = n_pad // NS

    @functools.partial(
        pl.kernel,
        mesh=_sc_mesh(),
        out_type=jax.ShapeDtypeStruct((NC * n_pad, h), jnp.float32),
        scratch_types=[
            pltpu.VMEM((CHUNK,), jnp.int32),
            pltpu.VMEM((CHUNK, h), jnp.float32),
            pltpu.VMEM_SHARED((n_pad, h), jnp.float32),
        ],
    )
    def deg_kernel(dst_hbm, ones_hbm, zero_hbm, out_hbm, didx, ones_v, acc):
        c = lax.axis_index("c")
        s = lax.axis_index("s")
        wid = c * NS + s
        r0 = s * rows
        pltpu.sync_copy(ones_hbm, ones_v)
        pltpu.sync_copy(zero_hbm.at[pl.ds(r0, rows), :], acc.at[pl.ds(r0, rows), :])
        plsc.subcore_barrier()

        def body(g, carry):
            e0 = wid * (nchunks * CHUNK) + g * CHUNK
            pltpu.sync_copy(dst_hbm.at[pl.ds(e0, CHUNK)], didx)
            pltpu.sync_copy(ones_v, acc.at[didx], add=True)
            return carry

        lax.fori_loop(0, nchunks, body, 0)
        plsc.subcore_barrier()
        pltpu.sync_copy(acc.at[pl.ds(r0, rows), :],
                        out_hbm.at[pl.ds(c * n_pad + r0, rows), :])

    return deg_kernel


@functools.lru_cache(maxsize=None)
def _make_sc_scatter(nchunks, n_pad, h):
    rows = n_pad // NS

    @functools.partial(
        pl.kernel,
        mesh=_sc_mesh(),
        out_type=jax.ShapeDtypeStruct((NC * n_pad, h), jnp.float32),
        scratch_types=[
            pltpu.VMEM((CHUNK,), jnp.int32),
            pltpu.VMEM((CHUNK,), jnp.int32),
            pltpu.VMEM((CHUNK, h), jnp.float32),
            pltpu.VMEM_SHARED((n_pad, h), jnp.float32),
            pltpu.SemaphoreType.DMA,
        ],
    )
    def scatter_kernel(y_hbm, src_hbm, dst_hbm, zero_hbm, out_hbm,
                       sidx, didx, gbuf, acc, sem):
        c = lax.axis_index("c")
        s = lax.axis_index("s")
        wid = c * NS + s
        r0 = s * rows
        pltpu.sync_copy(zero_hbm.at[pl.ds(r0, rows), :], acc.at[pl.ds(r0, rows), :])
        plsc.subcore_barrier()

        def body(g, carry):
            e0 = wid * (nchunks * CHUNK) + g * CHUNK
            pltpu.sync_copy(src_hbm.at[pl.ds(e0, CHUNK)], sidx)
            pltpu.sync_copy(dst_hbm.at[pl.ds(e0, CHUNK)], didx)
            pltpu.async_copy(y_hbm.at[sidx], gbuf, sem).wait()
            pltpu.sync_copy(gbuf, acc.at[didx], add=True)
            return carry

        lax.fori_loop(0, nchunks, body, 0)
        plsc.subcore_barrier()
        pltpu.sync_copy(acc.at[pl.ds(r0, rows), :],
                        out_hbm.at[pl.ds(c * n_pad + r0, rows), :])

    return scatter_kernel


def _tc_prep(x_pad, w1, degp):
    n_pad, f = x_pad.shape
    h = w1.shape[1]
    nb = n_pad // BM

    def body(x_ref, w_ref, d0_ref, d1_ref, y_ref, dinv_ref):
        deg = jnp.maximum(d0_ref[...][:, 0:1] + d1_ref[...][:, 0:1], 1.0)
        dinv = lax.rsqrt(deg)
        xw = jnp.dot(x_ref[...], w_ref[...],
                     preferred_element_type=jnp.float32)
        y_ref[...] = xw * dinv
        dinv_ref[...] = dinv

    return pl.pallas_call(
        body,
        grid=(nb,),
        in_specs=[
            pl.BlockSpec((BM, f), lambda i: (i, 0)),
            pl.BlockSpec((f, h), lambda i: (0, 0)),
            pl.BlockSpec((BM, h), lambda i: (i, 0)),
            pl.BlockSpec((BM, h), lambda i: (i + nb, 0)),
        ],
        out_specs=[
            pl.BlockSpec((BM, h), lambda i: (i, 0)),
            pl.BlockSpec((BM, 1), lambda i: (i, 0)),
        ],
        out_shape=[
            jax.ShapeDtypeStruct((n_pad, h), jnp.float32),
            jax.ShapeDtypeStruct((n_pad, 1), jnp.float32),
        ],
    )(x_pad, w1, degp, degp)


def _tc_mid(p, dinv, b1, w2):
    n_pad = p.shape[0] // NC
    h = p.shape[1]
    nb = n_pad // BM

    def body(p0_ref, p1_ref, dinv_ref, b_ref, w_ref, y2_ref):
        t = dinv_ref[...] * (p0_ref[...] + p1_ref[...]) + b_ref[...]
        hh = jnp.maximum(t, 0.0)
        y2_ref[...] = jnp.dot(hh, w_ref[...],
                              preferred_element_type=jnp.float32) * dinv_ref[...]

    return pl.pallas_call(
        body,
        grid=(nb,),
        in_specs=[
            pl.BlockSpec((BM, h), lambda i: (i, 0)),
            pl.BlockSpec((BM, h), lambda i: (i + nb, 0)),
            pl.BlockSpec((BM, 1), lambda i: (i, 0)),
            pl.BlockSpec((1, h), lambda i: (0, 0)),
            pl.BlockSpec((h, h), lambda i: (0, 0)),
        ],
        out_specs=pl.BlockSpec((BM, h), lambda i: (i, 0)),
        out_shape=jax.ShapeDtypeStruct((n_pad, h), jnp.float32),
    )(p, p, dinv, b1, w2)


def _tc_final(q, dinv, b2, batch_row, wlin, blin, g):
    n_pad = q.shape[0] // NC
    h = q.shape[1]
    nb = n_pad // BM

    def body(q0_ref, q1_ref, dinv_ref, b_ref, batch_ref, wlin_ref, blin_ref,
             out_ref, s_acc, c_acc):
        i = pl.program_id(0)

        @pl.when(i == 0)
        def _():
            s_acc[...] = jnp.zeros_like(s_acc)
            c_acc[...] = jnp.zeros_like(c_acc)

        t = dinv_ref[...] * (q0_ref[...] + q1_ref[...]) + b_ref[...]
        h2 = jnp.maximum(t, 0.0)
        seg = lax.broadcasted_iota(jnp.int32, (g, BM), 0)
        oh = (seg == batch_ref[...]).astype(jnp.float32)
        s_acc[...] += lax.dot_general(
            oh, h2, (((1,), (0,)), ((), ())),
            preferred_element_type=jnp.float32,
            precision=lax.Precision.HIGHEST)
        c_acc[...] += jnp.sum(oh, axis=1)[:, None]

        @pl.when(i == nb - 1)
        def _():
            pooled = s_acc[...] / jnp.maximum(c_acc[...], 1.0)
            out_ref[...] = jnp.dot(pooled, wlin_ref[...],
                                   preferred_element_type=jnp.float32) + blin_ref[...]

    return pl.pallas_call(
        body,
        grid=(nb,),
        in_specs=[
            pl.BlockSpec((BM, h), lambda i: (i, 0)),
            pl.BlockSpec((BM, h), lambda i: (i + nb, 0)),
            pl.BlockSpec((BM, 1), lambda i: (i, 0)),
            pl.BlockSpec((1, h), lambda i: (0, 0)),
            pl.BlockSpec((1, BM), lambda i: (0, i)),
            pl.BlockSpec((h, 1), lambda i: (0, 0)),
            pl.BlockSpec((1, 1), lambda i: (0, 0)),
        ],
        out_specs=pl.BlockSpec((g, 1), lambda i: (0, 0)),
        out_shape=jax.ShapeDtypeStruct((g, 1), jnp.float32),
        scratch_shapes=[
            pltpu.VMEM((g, h), jnp.float32),
            pltpu.VMEM((g, 1), jnp.float32),
        ],
    )(q, q, dinv, b2, batch_row, wlin, blin)


def _round_up(a, b):
    return -(-a // b) * b


def kernel(x, edge_index, batch, W1, b1, W2, b2, Wlin, blin):
    n, f = x.shape
    h = W1.shape[1]
    e = edge_index.shape[1]
    g = 64
    nt = NC * NS

    n_pad = _round_up(n + 1, BM)
    e2 = e + n
    per_tile = _round_up(-(-e2 // nt), CHUNK)
    ep = per_tile * nt
    nchunks = per_tile // CHUNK

    idt = jnp.int32
    loop = jnp.arange(n, dtype=idt)
    pad = ep - e2
    src_pad = jnp.concatenate(
        [edge_index[0].astype(idt), loop, jnp.zeros((pad,), idt)])
    dst_fill = n + jnp.arange(pad, dtype=idt) % (n_pad - n)
    dst_pad = jnp.concatenate(
        [edge_index[1].astype(idt), loop, dst_fill])

    x_pad = jnp.pad(x, ((0, n_pad - n), (0, 0)))
    batch_row = jnp.pad(batch.astype(idt), (0, n_pad - n), constant_values=g)[None, :]
    ones128 = jnp.ones((CHUNK, h), jnp.float32)
    z128 = jnp.zeros((n_pad, h), jnp.float32)

    degp = _make_sc_deg(nchunks, n_pad, h)(dst_pad, ones128, z128)
    y1, dinv = _tc_prep(x_pad, W1, degp)

    scat = _make_sc_scatter(nchunks, n_pad, h)
    p = scat(y1, src_pad, dst_pad, z128)
    y2 = _tc_mid(p, dinv, b1.reshape(1, h), W2)
    q = scat(y2, src_pad, dst_pad, z128)
    return _tc_final(q, dinv, b2.reshape(1, h), batch_row, Wlin, blin.reshape(1, 1), g)
